# Initial kernel scaffold; baseline (speedup 1.0000x reference)
import jax
import jax.numpy as jnp
from jax.experimental import pallas as pl


def kernel(x, edge_index, W1, b1, W2, b2):
    raise NotImplementedError("write your pallas kernel here")


# speedup vs baseline: 7.7674x; 7.7674x over previous
import functools

import jax
import jax.numpy as jnp
from jax import lax
from jax.experimental import pallas as pl
from jax.experimental.pallas import tpu as pltpu
from jax.experimental.pallas import tpu_sc as plsc

N_NODES = 10000
N_PAD = 10240
TILES = 32
BATCH = 128
NBT = 84
PER_TILE = NBT * BATCH
E_PAD = TILES * PER_TILE
NBUF = 4
IDX_ROWS = NBT + NBUF
NS = 16
ROWS_PER_SUB = N_PAD // NS

_SC_PARAMS = pltpu.CompilerParams(use_tc_tiling_on_sc=False)


def _make_sc_agg(d, nbuf, chunks):
    nc = NBT // chunks
    mesh = plsc.VectorSubcoreMesh(core_axis_name="c", subcore_axis_name="s")

    @functools.partial(
        pl.kernel,
        mesh=mesh,
        out_type=jax.ShapeDtypeStruct((2 * N_PAD, d), jnp.float32),
        scratch_types=[
            pltpu.VMEM((nc + nbuf, BATCH), jnp.int32),
            pltpu.VMEM((nc, BATCH), jnp.int32),
            pltpu.VMEM((nbuf, BATCH, d), jnp.float32),
            pltpu.VMEM_SHARED((N_PAD, d), jnp.float32),
        ] + [pltpu.SemaphoreType.DMA] * nbuf,
        compiler_params=_SC_PARAMS,
    )
    def agg(srcr_hbm, dstr_hbm, table_hbm, zeros_hbm, out_hbm,
            sidx, didx, gbuf, acc, *sems):
        c = lax.axis_index("c")
        s = lax.axis_index("s")
        wid = c * NS + s
        for r in range(nc, nc + nbuf):
            for k in range(BATCH // 16):
                sidx[r, pl.ds(k * 16, 16)] = jnp.zeros((16,), jnp.int32)
        pltpu.sync_copy(zeros_hbm.at[pl.ds(s * ROWS_PER_SUB, ROWS_PER_SUB)],
                        acc.at[pl.ds(s * ROWS_PER_SUB, ROWS_PER_SUB)])
        plsc.subcore_barrier()

        for ch in range(chunks):
            row0 = wid * NBT + ch * nc
            pltpu.sync_copy(srcr_hbm.at[pl.ds(row0, nc)],
                            sidx.at[pl.ds(0, nc)])
            pltpu.sync_copy(dstr_hbm.at[pl.ds(row0, nc)], didx)
            for b in range(nbuf):
                pltpu.async_copy(table_hbm.at[sidx.at[b]], gbuf.at[b],
                                 sems[b])

            def body(g, carry):
                for b in range(nbuf):
                    j = g * nbuf + b
                    pltpu.make_async_copy(table_hbm.at[sidx.at[b]],
                                          gbuf.at[b], sems[b]).wait()
                    pltpu.sync_copy(gbuf.at[b], acc.at[didx.at[j]], add=True)
                    pltpu.async_copy(table_hbm.at[sidx.at[j + nbuf]],
                                     gbuf.at[b], sems[b])
                return carry

            lax.fori_loop(0, nc // nbuf, body, 0)
            for b in range(nbuf):
                pltpu.make_async_copy(table_hbm.at[sidx.at[b]], gbuf.at[b],
                                      sems[b]).wait()
        plsc.subcore_barrier()
        pltpu.sync_copy(
            acc.at[pl.ds(s * ROWS_PER_SUB, ROWS_PER_SUB)],
            out_hbm.at[pl.ds(c * N_PAD + s * ROWS_PER_SUB, ROWS_PER_SUB)])

    return agg


def _make_sc_deg():
    mesh = plsc.VectorSubcoreMesh(core_axis_name="c", subcore_axis_name="s")

    @functools.partial(
        pl.kernel,
        mesh=mesh,
        out_type=jax.ShapeDtypeStruct((2 * N_PAD, 16), jnp.float32),
        scratch_types=[
            pltpu.VMEM((NBT, BATCH), jnp.int32),
            pltpu.VMEM((BATCH, 16), jnp.float32),
            pltpu.VMEM_SHARED((N_PAD, 16), jnp.float32),
            pltpu.SemaphoreType.DMA,
        ],
        compiler_params=_SC_PARAMS,
    )
    def deg(dstr_hbm, ones_hbm, zeros_hbm, out_hbm, didx, ones_v, acc, sem):
        c = lax.axis_index("c")
        s = lax.axis_index("s")
        wid = c * NS + s
        pltpu.sync_copy(ones_hbm, ones_v)
        pltpu.sync_copy(dstr_hbm.at[pl.ds(wid * NBT, NBT)], didx)
        pltpu.sync_copy(zeros_hbm.at[pl.ds(s * ROWS_PER_SUB, ROWS_PER_SUB)],
                        acc.at[pl.ds(s * ROWS_PER_SUB, ROWS_PER_SUB)])
        plsc.subcore_barrier()

        def fire(j, carry):
            pltpu.async_copy(ones_v, acc.at[didx.at[j]], sem, add=True)
            return carry

        lax.fori_loop(0, NBT, fire, 0)

        def drain(j, carry):
            pltpu.make_async_copy(ones_v, acc.at[didx.at[0]], sem).wait()
            return carry

        lax.fori_loop(0, NBT, drain, 0)
        plsc.subcore_barrier()
        pltpu.sync_copy(
            acc.at[pl.ds(s * ROWS_PER_SUB, ROWS_PER_SUB)],
            out_hbm.at[pl.ds(c * N_PAD + s * ROWS_PER_SUB, ROWS_PER_SUB)])

    return deg


_sc_deg = _make_sc_deg()
_sc_agg128 = _make_sc_agg(128, nbuf=2, chunks=2)
_sc_agg16 = _make_sc_agg(16, nbuf=4, chunks=1)

GRID = 8
RB = N_PAD // GRID


def _tc_layer1(xp, w1, degparts):
    def body(x_ref, w_ref, dp_ref, g1_ref, dinv_ref):
        deg = dp_ref[0] + dp_ref[1]
        dinv = lax.rsqrt(jnp.maximum(deg, 1.0))
        h = jnp.dot(x_ref[...], w_ref[...], preferred_element_type=jnp.float32)
        g1_ref[...] = dinv[:, 0:1] * h
        dinv_ref[...] = dinv

    return pl.pallas_call(
        body,
        grid=(GRID,),
        in_specs=[
            pl.BlockSpec((RB, 128), lambda i: (i, 0)),
            pl.BlockSpec((128, 128), lambda i: (0, 0)),
            pl.BlockSpec((2, RB, 16), lambda i: (0, i, 0)),
        ],
        out_specs=[
            pl.BlockSpec((RB, 128), lambda i: (i, 0)),
            pl.BlockSpec((RB, 16), lambda i: (i, 0)),
        ],
        out_shape=[
            jax.ShapeDtypeStruct((N_PAD, 128), jnp.float32),
            jax.ShapeDtypeStruct((N_PAD, 16), jnp.float32),
        ],
    )(xp, w1, degparts)


def _tc_layer2(aggparts, dinv16, b1r, w2p):
    def body(ap_ref, dinv_ref, b_ref, w_ref, g2_ref):
        agg = ap_ref[0] + ap_ref[1]
        dinv = dinv_ref[...]
        out1 = jnp.maximum(dinv[:, 0:1] * agg + b_ref[...], 0.0)
        g2_ref[...] = dinv * jnp.dot(out1, w_ref[...],
                                     preferred_element_type=jnp.float32)

    return pl.pallas_call(
        body,
        grid=(GRID,),
        in_specs=[
            pl.BlockSpec((2, RB, 128), lambda i: (0, i, 0)),
            pl.BlockSpec((RB, 16), lambda i: (i, 0)),
            pl.BlockSpec((1, 128), lambda i: (0, 0)),
            pl.BlockSpec((128, 16), lambda i: (0, 0)),
        ],
        out_specs=pl.BlockSpec((RB, 16), lambda i: (i, 0)),
        out_shape=jax.ShapeDtypeStruct((N_PAD, 16), jnp.float32),
    )(aggparts, dinv16, b1r, w2p)


def _tc_final(aggparts, dinv16, b2r):
    def body(ap_ref, dinv_ref, b_ref, o_ref):
        o_ref[...] = dinv_ref[...] * (ap_ref[0] + ap_ref[1]) + b_ref[...]

    return pl.pallas_call(
        body,
        grid=(GRID,),
        in_specs=[
            pl.BlockSpec((2, RB, 16), lambda i: (0, i, 0)),
            pl.BlockSpec((RB, 16), lambda i: (i, 0)),
            pl.BlockSpec((1, 16), lambda i: (0, 0)),
        ],
        out_specs=pl.BlockSpec((RB, 16), lambda i: (i, 0)),
        out_shape=jax.ShapeDtypeStruct((N_PAD, 16), jnp.float32),
    )(aggparts, dinv16, b2r)


def kernel(x, edge_index, W1, b1, W2, b2):
    n, din = x.shape
    dout = W2.shape[1]
    loopidx = jnp.arange(n, dtype=jnp.int32)
    pad_e = E_PAD - (edge_index.shape[1] + n)
    padv = jnp.full((pad_e,), n, jnp.int32)
    srcr = jnp.concatenate([edge_index[0], loopidx, padv]).reshape(-1, BATCH)
    dstr = jnp.concatenate([edge_index[1], loopidx, padv]).reshape(-1, BATCH)

    zeros16 = jnp.zeros((N_PAD, 16), jnp.float32)
    zeros128 = jnp.zeros((N_PAD, 128), jnp.float32)
    ones_b = jnp.ones((BATCH, 16), jnp.float32)

    degparts = _sc_deg(dstr, ones_b, zeros16).reshape(2, N_PAD, 16)

    xp = jnp.pad(x, ((0, N_PAD - n), (0, 0)))
    g1p, dinv16 = _tc_layer1(xp, W1, degparts)

    agg1 = _sc_agg128(srcr, dstr, g1p, zeros128).reshape(2, N_PAD, 128)

    w2p = jnp.pad(W2, ((0, 0), (0, 16 - dout)))
    g2p = _tc_layer2(agg1, dinv16, b1.reshape(1, din), w2p)

    agg2 = _sc_agg16(srcr, dstr, g2p, zeros16).reshape(2, N_PAD, 16)

    b2r = jnp.pad(b2, (0, 16 - dout)).reshape(1, 16)
    out16 = _tc_final(agg2, dinv16, b2r)
    return out16[:n, :dout]


# speedup vs baseline: 8.1827x; 1.0535x over previous
import functools

import jax
import jax.numpy as jnp
from jax import lax
from jax.experimental import pallas as pl
from jax.experimental.pallas import tpu as pltpu
from jax.experimental.pallas import tpu_sc as plsc

N_NODES = 10000
N_PAD = 10240
TILES = 32
BATCH = 128
NBT = 84
PER_TILE = NBT * BATCH
E_PAD = TILES * PER_TILE
NBUF = 4
IDX_ROWS = NBT + NBUF
NS = 16
ROWS_PER_SUB = N_PAD // NS

_SC_PARAMS = pltpu.CompilerParams(use_tc_tiling_on_sc=False)


def _make_sc_agg(d, nbuf, chunks):
    nc = NBT // chunks
    mesh = plsc.VectorSubcoreMesh(core_axis_name="c", subcore_axis_name="s")

    @functools.partial(
        pl.kernel,
        mesh=mesh,
        out_type=jax.ShapeDtypeStruct((2 * N_PAD, d), jnp.float32),
        scratch_types=[
            pltpu.VMEM((nc + nbuf, BATCH), jnp.int32),
            pltpu.VMEM((nc, BATCH), jnp.int32),
            pltpu.VMEM((nbuf, BATCH, d), jnp.float32),
            pltpu.VMEM_SHARED((N_PAD, d), jnp.float32),
        ] + [pltpu.SemaphoreType.DMA] * nbuf,
        compiler_params=_SC_PARAMS,
    )
    def agg(srcr_hbm, dstr_hbm, table_hbm, zeros_hbm, out_hbm,
            sidx, didx, gbuf, acc, *sems):
        c = lax.axis_index("c")
        s = lax.axis_index("s")
        wid = c * NS + s
        for r in range(nc, nc + nbuf):
            for k in range(BATCH // 16):
                sidx[r, pl.ds(k * 16, 16)] = jnp.zeros((16,), jnp.int32)
        pltpu.sync_copy(zeros_hbm.at[pl.ds(s * ROWS_PER_SUB, ROWS_PER_SUB)],
                        acc.at[pl.ds(s * ROWS_PER_SUB, ROWS_PER_SUB)])
        plsc.subcore_barrier()

        for ch in range(chunks):
            row0 = wid * NBT + ch * nc
            pltpu.sync_copy(srcr_hbm.at[pl.ds(row0, nc)],
                            sidx.at[pl.ds(0, nc)])
            pltpu.sync_copy(dstr_hbm.at[pl.ds(row0, nc)], didx)
            for b in range(nbuf):
                pltpu.async_copy(table_hbm.at[sidx.at[b]], gbuf.at[b],
                                 sems[b])

            def body(g, carry):
                for b in range(nbuf):
                    j = g * nbuf + b
                    pltpu.make_async_copy(table_hbm.at[sidx.at[b]],
                                          gbuf.at[b], sems[b]).wait()
                    pltpu.sync_copy(gbuf.at[b], acc.at[didx.at[j]], add=True)
                    pltpu.async_copy(table_hbm.at[sidx.at[j + nbuf]],
                                     gbuf.at[b], sems[b])
                return carry

            lax.fori_loop(0, nc // nbuf, body, 0)
            for b in range(nbuf):
                pltpu.make_async_copy(table_hbm.at[sidx.at[b]], gbuf.at[b],
                                      sems[b]).wait()
        plsc.subcore_barrier()
        pltpu.sync_copy(
            acc.at[pl.ds(s * ROWS_PER_SUB, ROWS_PER_SUB)],
            out_hbm.at[pl.ds(c * N_PAD + s * ROWS_PER_SUB, ROWS_PER_SUB)])

    return agg


def _make_sc_deg():
    mesh = plsc.VectorSubcoreMesh(core_axis_name="c", subcore_axis_name="s")

    @functools.partial(
        pl.kernel,
        mesh=mesh,
        out_type=jax.ShapeDtypeStruct((2 * N_PAD, 16), jnp.float32),
        scratch_types=[
            pltpu.VMEM((NBT, BATCH), jnp.int32),
            pltpu.VMEM((BATCH, 16), jnp.float32),
            pltpu.VMEM_SHARED((N_PAD, 16), jnp.float32),
            pltpu.SemaphoreType.DMA,
        ],
        compiler_params=_SC_PARAMS,
    )
    def deg(dstr_hbm, ones_hbm, zeros_hbm, out_hbm, didx, ones_v, acc, sem):
        c = lax.axis_index("c")
        s = lax.axis_index("s")
        wid = c * NS + s
        pltpu.sync_copy(ones_hbm, ones_v)
        pltpu.sync_copy(dstr_hbm.at[pl.ds(wid * NBT, NBT)], didx)
        pltpu.sync_copy(zeros_hbm.at[pl.ds(s * ROWS_PER_SUB, ROWS_PER_SUB)],
                        acc.at[pl.ds(s * ROWS_PER_SUB, ROWS_PER_SUB)])
        plsc.subcore_barrier()

        def fire(j, carry):
            pltpu.async_copy(ones_v, acc.at[didx.at[j]], sem, add=True)
            return carry

        lax.fori_loop(0, NBT, fire, 0)

        def drain(j, carry):
            pltpu.make_async_copy(ones_v, acc.at[didx.at[0]], sem).wait()
            return carry

        lax.fori_loop(0, NBT, drain, 0)
        plsc.subcore_barrier()
        pltpu.sync_copy(
            acc.at[pl.ds(s * ROWS_PER_SUB, ROWS_PER_SUB)],
            out_hbm.at[pl.ds(c * N_PAD + s * ROWS_PER_SUB, ROWS_PER_SUB)])

    return deg


def _make_sc_agg_wide(d):
    mesh = plsc.VectorSubcoreMesh(core_axis_name="c", subcore_axis_name="s")

    @functools.partial(
        pl.kernel,
        mesh=mesh,
        out_type=jax.ShapeDtypeStruct((2 * N_PAD, d), jnp.float32),
        scratch_types=[
            pltpu.VMEM((BATCH,), jnp.int32),
            pltpu.VMEM((BATCH,), jnp.int32),
            pltpu.VMEM((BATCH,), jnp.int32),
            pltpu.VMEM((BATCH,), jnp.int32),
            pltpu.VMEM((2, BATCH, d), jnp.float32),
            pltpu.VMEM_SHARED((N_PAD, d), jnp.float32),
            pltpu.SemaphoreType.DMA,
            pltpu.SemaphoreType.DMA,
        ],
        compiler_params=_SC_PARAMS,
    )
    def agg(srcf_hbm, dstf_hbm, table_hbm, zeros_hbm, out_hbm,
            sidx0, didx0, sidx1, didx1, gbuf, acc, sem0, sem1):
        sidxs = (sidx0, sidx1)
        didxs = (didx0, didx1)
        sems = (sem0, sem1)
        c = lax.axis_index("c")
        s = lax.axis_index("s")
        wid = c * NS + s
        ebase = wid * PER_TILE
        pltpu.sync_copy(zeros_hbm.at[pl.ds(s * ROWS_PER_SUB, ROWS_PER_SUB)],
                        acc.at[pl.ds(s * ROWS_PER_SUB, ROWS_PER_SUB)])
        plsc.subcore_barrier()

        for b in range(2):
            pltpu.sync_copy(srcf_hbm.at[pl.ds(ebase + b * BATCH, BATCH)],
                            sidxs[b])
            pltpu.sync_copy(dstf_hbm.at[pl.ds(ebase + b * BATCH, BATCH)],
                            didxs[b])
            pltpu.async_copy(table_hbm.at[sidxs[b]], gbuf.at[b], sems[b])

        def body(g, carry):
            for b in range(2):
                j = 2 * g + b
                pltpu.make_async_copy(table_hbm.at[sidxs[b]], gbuf.at[b],
                                      sems[b]).wait()
                pltpu.sync_copy(gbuf.at[b], acc.at[didxs[b]], add=True)
                nbase = ebase + (j + 2) * BATCH
                pltpu.sync_copy(srcf_hbm.at[pl.ds(nbase, BATCH)], sidxs[b])
                pltpu.sync_copy(dstf_hbm.at[pl.ds(nbase, BATCH)], didxs[b])
                pltpu.async_copy(table_hbm.at[sidxs[b]], gbuf.at[b], sems[b])
            return carry

        lax.fori_loop(0, NBT // 2, body, 0)
        for b in range(2):
            pltpu.make_async_copy(table_hbm.at[sidxs[b]], gbuf.at[b],
                                  sems[b]).wait()
        plsc.subcore_barrier()
        pltpu.sync_copy(
            acc.at[pl.ds(s * ROWS_PER_SUB, ROWS_PER_SUB)],
            out_hbm.at[pl.ds(c * N_PAD + s * ROWS_PER_SUB, ROWS_PER_SUB)])

    return agg


_sc_deg = _make_sc_deg()
_sc_agg128 = _make_sc_agg_wide(128)
_sc_agg16 = _make_sc_agg(16, nbuf=4, chunks=1)

GRID = 8
RB = N_PAD // GRID


def _tc_layer1(xp, w1, degparts):
    def body(x_ref, w_ref, dp_ref, g1_ref, dinv_ref):
        deg = dp_ref[0] + dp_ref[1]
        dinv = lax.rsqrt(jnp.maximum(deg, 1.0))
        h = jnp.dot(x_ref[...], w_ref[...], preferred_element_type=jnp.float32)
        g1_ref[...] = dinv[:, 0:1] * h
        dinv_ref[...] = dinv

    return pl.pallas_call(
        body,
        grid=(GRID,),
        in_specs=[
            pl.BlockSpec((RB, 128), lambda i: (i, 0)),
            pl.BlockSpec((128, 128), lambda i: (0, 0)),
            pl.BlockSpec((2, RB, 16), lambda i: (0, i, 0)),
        ],
        out_specs=[
            pl.BlockSpec((RB, 128), lambda i: (i, 0)),
            pl.BlockSpec((RB, 16), lambda i: (i, 0)),
        ],
        out_shape=[
            jax.ShapeDtypeStruct((N_PAD, 128), jnp.float32),
            jax.ShapeDtypeStruct((N_PAD, 16), jnp.float32),
        ],
    )(xp, w1, degparts)


def _tc_layer2(aggparts, dinv16, b1r, w2p):
    def body(ap_ref, dinv_ref, b_ref, w_ref, g2_ref):
        agg = ap_ref[0] + ap_ref[1]
        dinv = dinv_ref[...]
        out1 = jnp.maximum(dinv[:, 0:1] * agg + b_ref[...], 0.0)
        g2_ref[...] = dinv * jnp.dot(out1, w_ref[...],
                                     preferred_element_type=jnp.float32)

    return pl.pallas_call(
        body,
        grid=(GRID,),
        in_specs=[
            pl.BlockSpec((2, RB, 128), lambda i: (0, i, 0)),
            pl.BlockSpec((RB, 16), lambda i: (i, 0)),
            pl.BlockSpec((1, 128), lambda i: (0, 0)),
            pl.BlockSpec((128, 16), lambda i: (0, 0)),
        ],
        out_specs=pl.BlockSpec((RB, 16), lambda i: (i, 0)),
        out_shape=jax.ShapeDtypeStruct((N_PAD, 16), jnp.float32),
    )(aggparts, dinv16, b1r, w2p)


def _tc_final(aggparts, dinv16, b2r):
    def body(ap_ref, dinv_ref, b_ref, o_ref):
        o_ref[...] = dinv_ref[...] * (ap_ref[0] + ap_ref[1]) + b_ref[...]

    return pl.pallas_call(
        body,
        grid=(GRID,),
        in_specs=[
            pl.BlockSpec((2, RB, 16), lambda i: (0, i, 0)),
            pl.BlockSpec((RB, 16), lambda i: (i, 0)),
            pl.BlockSpec((1, 16), lambda i: (0, 0)),
        ],
        out_specs=pl.BlockSpec((RB, 16), lambda i: (i, 0)),
        out_shape=jax.ShapeDtypeStruct((N_PAD, 16), jnp.float32),
    )(aggparts, dinv16, b2r)


def kernel(x, edge_index, W1, b1, W2, b2):
    n, din = x.shape
    dout = W2.shape[1]
    loopidx = jnp.arange(n, dtype=jnp.int32)
    pad_e = E_PAD + 2 * BATCH - (edge_index.shape[1] + n)
    padv = jnp.full((pad_e,), n, jnp.int32)
    srcf = jnp.concatenate([edge_index[0], loopidx, padv])
    dstf = jnp.concatenate([edge_index[1], loopidx, padv])
    srcr = srcf[:E_PAD].reshape(-1, BATCH)
    dstr = dstf[:E_PAD].reshape(-1, BATCH)

    zeros16 = jnp.zeros((N_PAD, 16), jnp.float32)
    zeros128 = jnp.zeros((N_PAD, 128), jnp.float32)
    ones_b = jnp.ones((BATCH, 16), jnp.float32)

    degparts = _sc_deg(dstr, ones_b, zeros16).reshape(2, N_PAD, 16)

    xp = jnp.pad(x, ((0, N_PAD - n), (0, 0)))
    g1p, dinv16 = _tc_layer1(xp, W1, degparts)

    agg1 = _sc_agg128(srcf, dstf, g1p, zeros128).reshape(2, N_PAD, 128)

    w2p = jnp.pad(W2, ((0, 0), (0, 16 - dout)))
    g2p = _tc_layer2(agg1, dinv16, b1.reshape(1, din), w2p)

    agg2 = _sc_agg16(srcr, dstr, g2p, zeros16).reshape(2, N_PAD, 16)

    b2r = jnp.pad(b2, (0, 16 - dout)).reshape(1, 16)
    out16 = _tc_final(agg2, dinv16, b2r)
    return out16[:n, :dout]


# speedup vs baseline: 37.1302x; 4.5377x over previous
import functools

import jax
import jax.numpy as jnp
from jax import lax
from jax.experimental import pallas as pl
from jax.experimental.pallas import tpu as pltpu
from jax.experimental.pallas import tpu_sc as plsc

N_NODES = 10000
N_PAD = 10240
TILES = 32
BATCH = 128
NBT = 80
PER_TILE = NBT * BATCH
E_PAD = TILES * PER_TILE
NS = 16
ROWS_PER_SUB = N_PAD // NS

_SC_PARAMS = pltpu.CompilerParams(use_tc_tiling_on_sc=False)


def _make_sc_agg_ring(d, batch, nring, lead, chunks):
    nbt = PER_TILE // batch
    nc = nbt // chunks
    assert nc % nring == 0 and lead < nring
    assert ROWS_PER_SUB % batch == 0
    mesh = plsc.VectorSubcoreMesh(core_axis_name="c", subcore_axis_name="s")

    @functools.partial(
        pl.kernel,
        mesh=mesh,
        out_type=jax.ShapeDtypeStruct((2 * N_PAD, d), jnp.float32),
        scratch_types=[
            pltpu.VMEM((nc + lead, batch), jnp.int32),
            pltpu.VMEM((nc, batch), jnp.int32),
            pltpu.VMEM((nring, batch, d), jnp.float32),
            pltpu.VMEM_SHARED((N_PAD, d), jnp.float32),
        ] + [pltpu.SemaphoreType.DMA] * (2 * nring),
        compiler_params=_SC_PARAMS,
    )
    def agg(srcr_hbm, dstr_hbm, table_hbm, out_hbm,
            sidx, didx, gbuf, acc, *sems):
        gsems = sems[:nring]
        ssems = sems[nring:]
        c = lax.axis_index("c")
        s = lax.axis_index("s")
        wid = c * NS + s
        def zrow(r, carry):
            for k in range(d // 16):
                gbuf[0, r, pl.ds(k * 16, 16)] = jnp.zeros((16,), jnp.float32)
            return carry

        lax.fori_loop(0, batch, zrow, 0)
        for i in range(ROWS_PER_SUB // batch):
            pltpu.async_copy(
                gbuf.at[0],
                acc.at[pl.ds(s * ROWS_PER_SUB + i * batch, batch)], gsems[0])
        for i in range(ROWS_PER_SUB // batch):
            pltpu.make_async_copy(
                gbuf.at[0],
                acc.at[pl.ds(s * ROWS_PER_SUB, batch)], gsems[0]).wait()
        plsc.subcore_barrier()

        def visit(j, b):
            b2 = (b + lead) % nring
            pltpu.make_async_copy(table_hbm.at[sidx.at[b]], gbuf.at[b],
                                  gsems[b]).wait()
            pltpu.async_copy(gbuf.at[b], acc.at[didx.at[j]], ssems[b],
                             add=True)
            pltpu.async_copy(table_hbm.at[sidx.at[j + lead]], gbuf.at[b2],
                             gsems[b2])

        for ch in range(chunks):
            row0 = wid * nbt + ch * nc
            pltpu.sync_copy(srcr_hbm.at[pl.ds(row0, nc + lead)], sidx)
            pltpu.sync_copy(dstr_hbm.at[pl.ds(row0, nc)], didx)
            for jj in range(lead):
                pltpu.async_copy(table_hbm.at[sidx.at[jj]], gbuf.at[jj],
                                 gsems[jj])
            for jj in range(nring):
                b2 = (jj + lead) % nring
                if jj + lead >= nring:
                    pltpu.make_async_copy(gbuf.at[b2], acc.at[didx.at[0]],
                                          ssems[b2]).wait()
                visit(jj, jj)

            def body(g, carry):
                for bb in range(nring):
                    j = g * nring + bb
                    b2 = (bb + lead) % nring
                    pltpu.make_async_copy(gbuf.at[b2], acc.at[didx.at[0]],
                                          ssems[b2]).wait()
                    visit(j, bb)
                return carry

            lax.fori_loop(1, nc // nring, body, 0)
            for bb in range(lead, nring):
                pltpu.make_async_copy(gbuf.at[bb], acc.at[didx.at[0]],
                                      ssems[bb]).wait()
            for bb in range(lead):
                pltpu.make_async_copy(table_hbm.at[sidx.at[bb]], gbuf.at[bb],
                                      gsems[bb]).wait()
        plsc.subcore_barrier()
        pltpu.sync_copy(
            acc.at[pl.ds(s * ROWS_PER_SUB, ROWS_PER_SUB)],
            out_hbm.at[pl.ds(c * N_PAD + s * ROWS_PER_SUB, ROWS_PER_SUB)])

    return agg


def _make_sc_deg():
    mesh = plsc.VectorSubcoreMesh(core_axis_name="c", subcore_axis_name="s")

    @functools.partial(
        pl.kernel,
        mesh=mesh,
        out_type=jax.ShapeDtypeStruct((2 * N_PAD, 16), jnp.float32),
        scratch_types=[
            pltpu.VMEM((NBT, BATCH), jnp.int32),
            pltpu.VMEM((BATCH, 16), jnp.float32),
            pltpu.VMEM((BATCH, 16), jnp.float32),
            pltpu.VMEM_SHARED((N_PAD, 16), jnp.float32),
            pltpu.SemaphoreType.DMA,
        ],
        compiler_params=_SC_PARAMS,
    )
    def deg(dstr_hbm, out_hbm, didx, ones_v, zeros_v, acc, sem):
        c = lax.axis_index("c")
        s = lax.axis_index("s")
        wid = c * NS + s

        def fill(r, carry):
            ones_v[r, pl.ds(0, 16)] = jnp.ones((16,), jnp.float32)
            zeros_v[r, pl.ds(0, 16)] = jnp.zeros((16,), jnp.float32)
            return carry

        lax.fori_loop(0, BATCH, fill, 0)
        pltpu.sync_copy(dstr_hbm.at[pl.ds(wid * NBT, NBT)], didx)
        for i in range(ROWS_PER_SUB // BATCH):
            pltpu.async_copy(
                zeros_v,
                acc.at[pl.ds(s * ROWS_PER_SUB + i * BATCH, BATCH)], sem)
        for i in range(ROWS_PER_SUB // BATCH):
            pltpu.make_async_copy(
                zeros_v, acc.at[pl.ds(s * ROWS_PER_SUB, BATCH)], sem).wait()
        plsc.subcore_barrier()

        def fire(j, carry):
            pltpu.async_copy(ones_v, acc.at[didx.at[j]], sem, add=True)
            return carry

        lax.fori_loop(0, NBT, fire, 0)

        def drain(j, carry):
            pltpu.make_async_copy(ones_v, acc.at[didx.at[0]], sem).wait()
            return carry

        lax.fori_loop(0, NBT, drain, 0)
        plsc.subcore_barrier()
        pltpu.sync_copy(
            acc.at[pl.ds(s * ROWS_PER_SUB, ROWS_PER_SUB)],
            out_hbm.at[pl.ds(c * N_PAD + s * ROWS_PER_SUB, ROWS_PER_SUB)])

    return deg


_sc_deg = _make_sc_deg()
_sc_agg128 = _make_sc_agg_ring(128, batch=64, nring=4, lead=2, chunks=2)
_sc_agg16 = _make_sc_agg_ring(16, batch=128, nring=4, lead=2, chunks=1)

GRID = 8
RB = N_PAD // GRID


def _tc_matmul1(xp, w1):
    def body(x_ref, w_ref, h_ref):
        h_ref[...] = jnp.dot(x_ref[...], w_ref[...],
                             preferred_element_type=jnp.float32)

    return pl.pallas_call(
        body,
        grid=(GRID,),
        in_specs=[
            pl.BlockSpec((RB, 128), lambda i: (i, 0)),
            pl.BlockSpec((128, 128), lambda i: (0, 0)),
        ],
        out_specs=pl.BlockSpec((RB, 128), lambda i: (i, 0)),
        out_shape=jax.ShapeDtypeStruct((N_PAD, 128), jnp.float32),
    )(xp, w1)


def _tc_scale1(h1, degparts):
    def body(h_ref, dp_ref, g1_ref, dinv_ref):
        dinv = lax.rsqrt(dp_ref[0] + dp_ref[1] + 1.0)
        g1_ref[...] = dinv[:, 0:1] * h_ref[...]
        dinv_ref[...] = dinv

    return pl.pallas_call(
        body,
        grid=(GRID,),
        in_specs=[
            pl.BlockSpec((RB, 128), lambda i: (i, 0)),
            pl.BlockSpec((2, RB, 16), lambda i: (0, i, 0)),
        ],
        out_specs=[
            pl.BlockSpec((RB, 128), lambda i: (i, 0)),
            pl.BlockSpec((RB, 16), lambda i: (i, 0)),
        ],
        out_shape=[
            jax.ShapeDtypeStruct((N_PAD, 128), jnp.float32),
            jax.ShapeDtypeStruct((N_PAD, 16), jnp.float32),
        ],
    )(h1, degparts)


def _tc_layer2(aggparts, dinv16, g1p, b1r, w2p):
    def body(ap_ref, dinv_ref, g1_ref, b_ref, w_ref, g2_ref):
        dinv = dinv_ref[...]
        tot = ap_ref[0] + ap_ref[1] + g1_ref[...]
        out1 = jnp.maximum(dinv[:, 0:1] * tot + b_ref[...], 0.0)
        g2_ref[...] = dinv * jnp.dot(out1, w_ref[...],
                                     preferred_element_type=jnp.float32)

    return pl.pallas_call(
        body,
        grid=(GRID,),
        in_specs=[
            pl.BlockSpec((2, RB, 128), lambda i: (0, i, 0)),
            pl.BlockSpec((RB, 16), lambda i: (i, 0)),
            pl.BlockSpec((RB, 128), lambda i: (i, 0)),
            pl.BlockSpec((1, 128), lambda i: (0, 0)),
            pl.BlockSpec((128, 16), lambda i: (0, 0)),
        ],
        out_specs=pl.BlockSpec((RB, 16), lambda i: (i, 0)),
        out_shape=jax.ShapeDtypeStruct((N_PAD, 16), jnp.float32),
    )(aggparts, dinv16, g1p, b1r, w2p)


def _tc_final(aggparts, dinv16, g2p, b2r):
    def body(ap_ref, dinv_ref, g2_ref, b_ref, o_ref):
        o_ref[...] = (dinv_ref[...] * (ap_ref[0] + ap_ref[1] + g2_ref[...])
                      + b_ref[...])

    return pl.pallas_call(
        body,
        grid=(GRID,),
        in_specs=[
            pl.BlockSpec((2, RB, 16), lambda i: (0, i, 0)),
            pl.BlockSpec((RB, 16), lambda i: (i, 0)),
            pl.BlockSpec((RB, 16), lambda i: (i, 0)),
            pl.BlockSpec((1, 16), lambda i: (0, 0)),
        ],
        out_specs=pl.BlockSpec((RB, 16), lambda i: (i, 0)),
        out_shape=jax.ShapeDtypeStruct((N_PAD, 16), jnp.float32),
    )(aggparts, dinv16, g2p, b2r)


def kernel(x, edge_index, W1, b1, W2, b2):
    n, din = x.shape
    dout = W2.shape[1]
    pad_e = E_PAD + 2 * BATCH - edge_index.shape[1]
    padv = (n + jnp.arange(pad_e, dtype=jnp.int32) % (N_PAD - n))
    srcf = jnp.concatenate([edge_index[0], padv])
    dstf = jnp.concatenate([edge_index[1], padv])
    srcr = srcf.reshape(-1, BATCH)
    dstr = dstf.reshape(-1, BATCH)
    srcr64 = srcf.reshape(-1, 64)
    dstr64 = dstf.reshape(-1, 64)

    degparts = _sc_deg(dstr).reshape(2, N_PAD, 16)

    xp = jnp.pad(x, ((0, N_PAD - n), (0, 0)))
    h1 = _tc_matmul1(xp, W1)
    g1p, dinv16 = _tc_scale1(h1, degparts)

    agg1 = _sc_agg128(srcr64, dstr64, g1p).reshape(2, N_PAD, 128)

    w2p = jnp.pad(W2, ((0, 0), (0, 16 - dout)))
    g2p = _tc_layer2(agg1, dinv16, g1p, b1.reshape(1, din), w2p)

    agg2 = _sc_agg16(srcr, dstr, g2p).reshape(2, N_PAD, 16)

    b2r = jnp.pad(b2, (0, 16 - dout)).reshape(1, 16)
    out16 = _tc_final(agg2, dinv16, g2p, b2r)
    return out16[:n, :dout]


# speedup vs baseline: 41.0962x; 1.1068x over previous
import functools

import jax
import jax.numpy as jnp
from jax import lax
from jax.experimental import pallas as pl
from jax.experimental.pallas import tpu as pltpu
from jax.experimental.pallas import tpu_sc as plsc

N_NODES = 10000
N_PAD = 10240
TILES = 32
BATCH = 128
NBT = 80
PER_TILE = NBT * BATCH
E_PAD = TILES * PER_TILE
NS = 16
ROWS_PER_SUB = N_PAD // NS

_SC_PARAMS = pltpu.CompilerParams(use_tc_tiling_on_sc=False)


def _make_sc_agg_ring(d, batch, nring, lead, chunks):
    nbt = PER_TILE // batch
    nc = nbt // chunks
    assert nc % nring == 0 and lead < nring
    assert ROWS_PER_SUB % batch == 0
    mesh = plsc.VectorSubcoreMesh(core_axis_name="c", subcore_axis_name="s")

    @functools.partial(
        pl.kernel,
        mesh=mesh,
        out_type=jax.ShapeDtypeStruct((2 * N_PAD, d), jnp.float32),
        scratch_types=[
            pltpu.VMEM((nc + lead, batch), jnp.int32),
            pltpu.VMEM((nc, batch), jnp.int32),
            pltpu.VMEM((nring, batch, d), jnp.float32),
            pltpu.VMEM_SHARED((N_PAD, d), jnp.float32),
        ] + [pltpu.SemaphoreType.DMA] * (2 * nring),
        compiler_params=_SC_PARAMS,
    )
    def agg(srcr_hbm, dstr_hbm, table_hbm, out_hbm,
            sidx, didx, gbuf, acc, *sems):
        gsems = sems[:nring]
        ssems = sems[nring:]
        c = lax.axis_index("c")
        s = lax.axis_index("s")
        wid = c * NS + s
        def zrow(r, carry):
            for k in range(d // 16):
                gbuf[0, r, pl.ds(k * 16, 16)] = jnp.zeros((16,), jnp.float32)
            return carry

        lax.fori_loop(0, batch, zrow, 0)
        for i in range(ROWS_PER_SUB // batch):
            pltpu.async_copy(
                gbuf.at[0],
                acc.at[pl.ds(s * ROWS_PER_SUB + i * batch, batch)], gsems[0])
        for i in range(ROWS_PER_SUB // batch):
            pltpu.make_async_copy(
                gbuf.at[0],
                acc.at[pl.ds(s * ROWS_PER_SUB, batch)], gsems[0]).wait()
        plsc.subcore_barrier()

        def visit(j, b):
            b2 = (b + lead) % nring
            pltpu.make_async_copy(table_hbm.at[sidx.at[b]], gbuf.at[b],
                                  gsems[b]).wait()
            pltpu.async_copy(gbuf.at[b], acc.at[didx.at[j]], ssems[b],
                             add=True)
            pltpu.async_copy(table_hbm.at[sidx.at[j + lead]], gbuf.at[b2],
                             gsems[b2])

        for ch in range(chunks):
            row0 = wid * nbt + ch * nc
            pltpu.sync_copy(srcr_hbm.at[pl.ds(row0, nc + lead)], sidx)
            pltpu.sync_copy(dstr_hbm.at[pl.ds(row0, nc)], didx)
            for jj in range(lead):
                pltpu.async_copy(table_hbm.at[sidx.at[jj]], gbuf.at[jj],
                                 gsems[jj])
            for jj in range(nring):
                b2 = (jj + lead) % nring
                if jj + lead >= nring:
                    pltpu.make_async_copy(gbuf.at[b2], acc.at[didx.at[0]],
                                          ssems[b2]).wait()
                visit(jj, jj)

            def body(g, carry):
                for bb in range(nring):
                    j = g * nring + bb
                    b2 = (bb + lead) % nring
                    pltpu.make_async_copy(gbuf.at[b2], acc.at[didx.at[0]],
                                          ssems[b2]).wait()
                    visit(j, bb)
                return carry

            lax.fori_loop(1, nc // nring, body, 0)
            for bb in range(lead, nring):
                pltpu.make_async_copy(gbuf.at[bb], acc.at[didx.at[0]],
                                      ssems[bb]).wait()
            for bb in range(lead):
                pltpu.make_async_copy(table_hbm.at[sidx.at[bb]], gbuf.at[bb],
                                      gsems[bb]).wait()
        plsc.subcore_barrier()
        pltpu.sync_copy(
            acc.at[pl.ds(s * ROWS_PER_SUB, ROWS_PER_SUB)],
            out_hbm.at[pl.ds(c * N_PAD + s * ROWS_PER_SUB, ROWS_PER_SUB)])

    return agg


def _make_sc_deg():
    mesh = plsc.VectorSubcoreMesh(core_axis_name="c", subcore_axis_name="s")

    @functools.partial(
        pl.kernel,
        mesh=mesh,
        out_type=jax.ShapeDtypeStruct((2 * N_PAD, 16), jnp.float32),
        scratch_types=[
            pltpu.VMEM((NBT, BATCH), jnp.int32),
            pltpu.VMEM((BATCH, 16), jnp.float32),
            pltpu.VMEM((BATCH, 16), jnp.float32),
            pltpu.VMEM_SHARED((N_PAD, 16), jnp.float32),
            pltpu.SemaphoreType.DMA,
        ],
        compiler_params=_SC_PARAMS,
    )
    def deg(dstr_hbm, out_hbm, didx, ones_v, zeros_v, acc, sem):
        c = lax.axis_index("c")
        s = lax.axis_index("s")
        wid = c * NS + s

        def fill(r, carry):
            ones_v[r, pl.ds(0, 16)] = jnp.ones((16,), jnp.float32)
            zeros_v[r, pl.ds(0, 16)] = jnp.zeros((16,), jnp.float32)
            return carry

        lax.fori_loop(0, BATCH, fill, 0)
        pltpu.sync_copy(dstr_hbm.at[pl.ds(wid * NBT, NBT)], didx)
        for i in range(ROWS_PER_SUB // BATCH):
            pltpu.async_copy(
                zeros_v,
                acc.at[pl.ds(s * ROWS_PER_SUB + i * BATCH, BATCH)], sem)
        for i in range(ROWS_PER_SUB // BATCH):
            pltpu.make_async_copy(
                zeros_v, acc.at[pl.ds(s * ROWS_PER_SUB, BATCH)], sem).wait()
        plsc.subcore_barrier()

        def fire(j, carry):
            pltpu.async_copy(ones_v, acc.at[didx.at[j]], sem, add=True)
            return carry

        lax.fori_loop(0, NBT, fire, 0)

        def drain(j, carry):
            pltpu.make_async_copy(ones_v, acc.at[didx.at[0]], sem).wait()
            return carry

        lax.fori_loop(0, NBT, drain, 0)
        plsc.subcore_barrier()
        pltpu.sync_copy(
            acc.at[pl.ds(s * ROWS_PER_SUB, ROWS_PER_SUB)],
            out_hbm.at[pl.ds(c * N_PAD + s * ROWS_PER_SUB, ROWS_PER_SUB)])

    return deg


_sc_deg = _make_sc_deg()
_sc_agg128 = _make_sc_agg_ring(128, batch=64, nring=5, lead=3, chunks=4)
_sc_agg16 = _make_sc_agg_ring(16, batch=128, nring=8, lead=4, chunks=1)

GRID = 8
RB = N_PAD // GRID


def _tc_matmul1(xp, w1):
    def body(x_ref, w_ref, h_ref):
        h_ref[...] = jnp.dot(x_ref[...], w_ref[...],
                             preferred_element_type=jnp.float32)

    return pl.pallas_call(
        body,
        grid=(GRID,),
        in_specs=[
            pl.BlockSpec((RB, 128), lambda i: (i, 0)),
            pl.BlockSpec((128, 128), lambda i: (0, 0)),
        ],
        out_specs=pl.BlockSpec((RB, 128), lambda i: (i, 0)),
        out_shape=jax.ShapeDtypeStruct((N_PAD, 128), jnp.float32),
    )(xp, w1)


def _tc_scale1(h1, degparts):
    def body(h_ref, dp_ref, g1_ref, dinv_ref):
        dinv = lax.rsqrt(dp_ref[0] + dp_ref[1] + 1.0)
        g1_ref[...] = dinv[:, 0:1] * h_ref[...]
        dinv_ref[...] = dinv

    return pl.pallas_call(
        body,
        grid=(GRID,),
        in_specs=[
            pl.BlockSpec((RB, 128), lambda i: (i, 0)),
            pl.BlockSpec((2, RB, 16), lambda i: (0, i, 0)),
        ],
        out_specs=[
            pl.BlockSpec((RB, 128), lambda i: (i, 0)),
            pl.BlockSpec((RB, 16), lambda i: (i, 0)),
        ],
        out_shape=[
            jax.ShapeDtypeStruct((N_PAD, 128), jnp.float32),
            jax.ShapeDtypeStruct((N_PAD, 16), jnp.float32),
        ],
    )(h1, degparts)


def _tc_layer2(aggparts, dinv16, g1p, b1r, w2p):
    def body(ap_ref, dinv_ref, g1_ref, b_ref, w_ref, g2_ref):
        dinv = dinv_ref[...]
        tot = ap_ref[0] + ap_ref[1] + g1_ref[...]
        out1 = jnp.maximum(dinv[:, 0:1] * tot + b_ref[...], 0.0)
        g2_ref[...] = dinv * jnp.dot(out1, w_ref[...],
                                     preferred_element_type=jnp.float32)

    return pl.pallas_call(
        body,
        grid=(GRID,),
        in_specs=[
            pl.BlockSpec((2, RB, 128), lambda i: (0, i, 0)),
            pl.BlockSpec((RB, 16), lambda i: (i, 0)),
            pl.BlockSpec((RB, 128), lambda i: (i, 0)),
            pl.BlockSpec((1, 128), lambda i: (0, 0)),
            pl.BlockSpec((128, 16), lambda i: (0, 0)),
        ],
        out_specs=pl.BlockSpec((RB, 16), lambda i: (i, 0)),
        out_shape=jax.ShapeDtypeStruct((N_PAD, 16), jnp.float32),
    )(aggparts, dinv16, g1p, b1r, w2p)


def _tc_final(aggparts, dinv16, g2p, b2r):
    def body(ap_ref, dinv_ref, g2_ref, b_ref, o_ref):
        o_ref[...] = (dinv_ref[...] * (ap_ref[0] + ap_ref[1] + g2_ref[...])
                      + b_ref[...])

    return pl.pallas_call(
        body,
        grid=(GRID,),
        in_specs=[
            pl.BlockSpec((2, RB, 16), lambda i: (0, i, 0)),
            pl.BlockSpec((RB, 16), lambda i: (i, 0)),
            pl.BlockSpec((RB, 16), lambda i: (i, 0)),
            pl.BlockSpec((1, 16), lambda i: (0, 0)),
        ],
        out_specs=pl.BlockSpec((RB, 16), lambda i: (i, 0)),
        out_shape=jax.ShapeDtypeStruct((N_PAD, 16), jnp.float32),
    )(aggparts, dinv16, g2p, b2r)


def kernel(x, edge_index, W1, b1, W2, b2):
    n, din = x.shape
    dout = W2.shape[1]
    pad_e = E_PAD + 2 * BATCH - edge_index.shape[1]
    padv = (n + jnp.arange(pad_e, dtype=jnp.int32) % (N_PAD - n))
    srcf = jnp.concatenate([edge_index[0], padv])
    dstf = jnp.concatenate([edge_index[1], padv])
    srcr = srcf.reshape(-1, BATCH)
    dstr = dstf.reshape(-1, BATCH)
    srcr64 = srcf.reshape(-1, 64)
    dstr64 = dstf.reshape(-1, 64)

    degparts = _sc_deg(dstr).reshape(2, N_PAD, 16)

    xp = jnp.pad(x, ((0, N_PAD - n), (0, 0)))
    h1 = _tc_matmul1(xp, W1)
    g1p, dinv16 = _tc_scale1(h1, degparts)

    agg1 = _sc_agg128(srcr64, dstr64, g1p).reshape(2, N_PAD, 128)

    w2p = jnp.pad(W2, ((0, 0), (0, 16 - dout)))
    g2p = _tc_layer2(agg1, dinv16, g1p, b1.reshape(1, din), w2p)

    agg2 = _sc_agg16(srcr, dstr, g2p).reshape(2, N_PAD, 16)

    b2r = jnp.pad(b2, (0, 16 - dout)).reshape(1, 16)
    out16 = _tc_final(agg2, dinv16, g2p, b2r)
    return out16[:n, :dout]


# speedup vs baseline: 42.0363x; 1.0229x over previous
import functools

import jax
import jax.numpy as jnp
from jax import lax
from jax.experimental import pallas as pl
from jax.experimental.pallas import tpu as pltpu
from jax.experimental.pallas import tpu_sc as plsc

N_NODES = 10000
N_PAD = 10240
TILES = 32
BATCH = 128
NBT = 80
PER_TILE = NBT * BATCH
E_PAD = TILES * PER_TILE
NS = 16
ROWS_PER_SUB = N_PAD // NS
ROWS_OUT = N_NODES // NS

_SC_PARAMS = pltpu.CompilerParams(use_tc_tiling_on_sc=False)


def _make_sc_agg_ring(d, batch, nring, lead, chunks):
    nbt = PER_TILE // batch
    nc = nbt // chunks
    assert nc % nring == 0 and lead < nring
    assert ROWS_PER_SUB % batch == 0
    mesh = plsc.VectorSubcoreMesh(core_axis_name="c", subcore_axis_name="s")

    @functools.partial(
        pl.kernel,
        mesh=mesh,
        out_type=jax.ShapeDtypeStruct((2 * N_NODES, d), jnp.float32),
        scratch_types=[
            pltpu.VMEM((nc + lead, batch), jnp.int32),
            pltpu.VMEM((nc, batch), jnp.int32),
            pltpu.VMEM((nring, batch, d), jnp.float32),
            pltpu.VMEM_SHARED((N_PAD, d), jnp.float32),
        ] + [pltpu.SemaphoreType.DMA] * (2 * nring),
        compiler_params=_SC_PARAMS,
    )
    def agg(srcr_hbm, dstr_hbm, table_hbm, out_hbm,
            sidx, didx, gbuf, acc, *sems):
        gsems = sems[:nring]
        ssems = sems[nring:]
        c = lax.axis_index("c")
        s = lax.axis_index("s")
        wid = c * NS + s
        def zrow(r, carry):
            for k in range(d // 16):
                gbuf[0, r, pl.ds(k * 16, 16)] = jnp.zeros((16,), jnp.float32)
            return carry

        lax.fori_loop(0, batch, zrow, 0)
        for i in range(ROWS_PER_SUB // batch):
            pltpu.async_copy(
                gbuf.at[0],
                acc.at[pl.ds(s * ROWS_PER_SUB + i * batch, batch)], gsems[0])
        for i in range(ROWS_PER_SUB // batch):
            pltpu.make_async_copy(
                gbuf.at[0],
                acc.at[pl.ds(s * ROWS_PER_SUB, batch)], gsems[0]).wait()
        plsc.subcore_barrier()

        def visit(j, b):
            b2 = (b + lead) % nring
            pltpu.make_async_copy(table_hbm.at[sidx.at[b]], gbuf.at[b],
                                  gsems[b]).wait()
            pltpu.async_copy(gbuf.at[b], acc.at[didx.at[j]], ssems[b],
                             add=True)
            pltpu.async_copy(table_hbm.at[sidx.at[j + lead]], gbuf.at[b2],
                             gsems[b2])

        for ch in range(chunks):
            row0 = wid * nbt + ch * nc
            pltpu.sync_copy(srcr_hbm.at[pl.ds(row0, nc + lead)], sidx)
            pltpu.sync_copy(dstr_hbm.at[pl.ds(row0, nc)], didx)
            for jj in range(lead):
                pltpu.async_copy(table_hbm.at[sidx.at[jj]], gbuf.at[jj],
                                 gsems[jj])
            for jj in range(nring):
                b2 = (jj + lead) % nring
                if jj + lead >= nring:
                    pltpu.make_async_copy(gbuf.at[b2], acc.at[didx.at[0]],
                                          ssems[b2]).wait()
                visit(jj, jj)

            def body(g, carry):
                for bb in range(nring):
                    j = g * nring + bb
                    b2 = (bb + lead) % nring
                    pltpu.make_async_copy(gbuf.at[b2], acc.at[didx.at[0]],
                                          ssems[b2]).wait()
                    visit(j, bb)
                return carry

            lax.fori_loop(1, nc // nring, body, 0)
            for bb in range(lead, nring):
                pltpu.make_async_copy(gbuf.at[bb], acc.at[didx.at[0]],
                                      ssems[bb]).wait()
            for bb in range(lead):
                pltpu.make_async_copy(table_hbm.at[sidx.at[bb]], gbuf.at[bb],
                                      gsems[bb]).wait()
        plsc.subcore_barrier()
        pltpu.sync_copy(
            acc.at[pl.ds(s * ROWS_OUT, ROWS_OUT)],
            out_hbm.at[pl.ds(c * N_NODES + s * ROWS_OUT, ROWS_OUT)])

    return agg


def _make_sc_deg():
    mesh = plsc.VectorSubcoreMesh(core_axis_name="c", subcore_axis_name="s")

    @functools.partial(
        pl.kernel,
        mesh=mesh,
        out_type=jax.ShapeDtypeStruct((2 * N_NODES, 16), jnp.float32),
        scratch_types=[
            pltpu.VMEM((NBT, BATCH), jnp.int32),
            pltpu.VMEM((BATCH, 16), jnp.float32),
            pltpu.VMEM((BATCH, 16), jnp.float32),
            pltpu.VMEM_SHARED((N_PAD, 16), jnp.float32),
            pltpu.SemaphoreType.DMA,
        ],
        compiler_params=_SC_PARAMS,
    )
    def deg(dstr_hbm, out_hbm, didx, ones_v, zeros_v, acc, sem):
        c = lax.axis_index("c")
        s = lax.axis_index("s")
        wid = c * NS + s

        def fill(r, carry):
            ones_v[r, pl.ds(0, 16)] = jnp.ones((16,), jnp.float32)
            zeros_v[r, pl.ds(0, 16)] = jnp.zeros((16,), jnp.float32)
            return carry

        lax.fori_loop(0, BATCH, fill, 0)
        pltpu.sync_copy(dstr_hbm.at[pl.ds(wid * NBT, NBT)], didx)
        for i in range(ROWS_PER_SUB // BATCH):
            pltpu.async_copy(
                zeros_v,
                acc.at[pl.ds(s * ROWS_PER_SUB + i * BATCH, BATCH)], sem)
        for i in range(ROWS_PER_SUB // BATCH):
            pltpu.make_async_copy(
                zeros_v, acc.at[pl.ds(s * ROWS_PER_SUB, BATCH)], sem).wait()
        plsc.subcore_barrier()

        def fire(j, carry):
            pltpu.async_copy(ones_v, acc.at[didx.at[j]], sem, add=True)
            return carry

        lax.fori_loop(0, NBT, fire, 0)

        def drain(j, carry):
            pltpu.make_async_copy(ones_v, acc.at[didx.at[0]], sem).wait()
            return carry

        lax.fori_loop(0, NBT, drain, 0)
        plsc.subcore_barrier()
        pltpu.sync_copy(
            acc.at[pl.ds(s * ROWS_OUT, ROWS_OUT)],
            out_hbm.at[pl.ds(c * N_NODES + s * ROWS_OUT, ROWS_OUT)])

    return deg


_sc_deg = _make_sc_deg()
_sc_agg128 = _make_sc_agg_ring(128, batch=64, nring=5, lead=3, chunks=4)
_sc_agg16 = _make_sc_agg_ring(16, batch=128, nring=8, lead=4, chunks=1)

GRID = 5
RB = N_NODES // GRID


def _tc_matmul1(xp, w1):
    def body(x_ref, w_ref, h_ref):
        h_ref[...] = jnp.dot(x_ref[...], w_ref[...],
                             preferred_element_type=jnp.float32)

    return pl.pallas_call(
        body,
        grid=(GRID,),
        in_specs=[
            pl.BlockSpec((RB, 128), lambda i: (i, 0)),
            pl.BlockSpec((128, 128), lambda i: (0, 0)),
        ],
        out_specs=pl.BlockSpec((RB, 128), lambda i: (i, 0)),
        out_shape=jax.ShapeDtypeStruct((N_NODES, 128), jnp.float32),
    )(xp, w1)


def _tc_scale1(h1, degparts):
    def body(h_ref, dp_ref, g1_ref, dinv_ref):
        dinv = lax.rsqrt(dp_ref[0] + dp_ref[1] + 1.0)
        g1_ref[...] = dinv[:, 0:1] * h_ref[...]
        dinv_ref[...] = dinv

    return pl.pallas_call(
        body,
        grid=(GRID,),
        in_specs=[
            pl.BlockSpec((RB, 128), lambda i: (i, 0)),
            pl.BlockSpec((2, RB, 16), lambda i: (0, i, 0)),
        ],
        out_specs=[
            pl.BlockSpec((RB, 128), lambda i: (i, 0)),
            pl.BlockSpec((RB, 16), lambda i: (i, 0)),
        ],
        out_shape=[
            jax.ShapeDtypeStruct((N_NODES, 128), jnp.float32),
            jax.ShapeDtypeStruct((N_NODES, 16), jnp.float32),
        ],
    )(h1, degparts)


def _tc_layer2(aggparts, dinv16, g1p, b1r, w2p):
    def body(ap_ref, dinv_ref, g1_ref, b_ref, w_ref, g2_ref):
        dinv = dinv_ref[...]
        tot = ap_ref[0] + ap_ref[1] + g1_ref[...]
        out1 = jnp.maximum(dinv[:, 0:1] * tot + b_ref[...], 0.0)
        g2_ref[...] = dinv * jnp.dot(out1, w_ref[...],
                                     preferred_element_type=jnp.float32)

    return pl.pallas_call(
        body,
        grid=(GRID,),
        in_specs=[
            pl.BlockSpec((2, RB, 128), lambda i: (0, i, 0)),
            pl.BlockSpec((RB, 16), lambda i: (i, 0)),
            pl.BlockSpec((RB, 128), lambda i: (i, 0)),
            pl.BlockSpec((1, 128), lambda i: (0, 0)),
            pl.BlockSpec((128, 16), lambda i: (0, 0)),
        ],
        out_specs=pl.BlockSpec((RB, 16), lambda i: (i, 0)),
        out_shape=jax.ShapeDtypeStruct((N_NODES, 16), jnp.float32),
    )(aggparts, dinv16, g1p, b1r, w2p)


def _tc_final(aggparts, dinv16, g2p, b2r):
    def body(ap_ref, dinv_ref, g2_ref, b_ref, o_ref):
        o_ref[...] = (dinv_ref[...] * (ap_ref[0] + ap_ref[1] + g2_ref[...])
                      + b_ref[...])

    return pl.pallas_call(
        body,
        grid=(GRID,),
        in_specs=[
            pl.BlockSpec((2, RB, 16), lambda i: (0, i, 0)),
            pl.BlockSpec((RB, 16), lambda i: (i, 0)),
            pl.BlockSpec((RB, 16), lambda i: (i, 0)),
            pl.BlockSpec((1, 16), lambda i: (0, 0)),
        ],
        out_specs=pl.BlockSpec((RB, 16), lambda i: (i, 0)),
        out_shape=jax.ShapeDtypeStruct((N_NODES, 16), jnp.float32),
    )(aggparts, dinv16, g2p, b2r)


def kernel(x, edge_index, W1, b1, W2, b2):
    n, din = x.shape
    dout = W2.shape[1]
    pad_e = E_PAD + 2 * BATCH - edge_index.shape[1]
    padc = jnp.arange(pad_e, dtype=jnp.int32) % (N_PAD - n)
    srcf = jnp.concatenate([edge_index[0], padc])
    dstf = jnp.concatenate([edge_index[1], n + padc])
    srcr = srcf.reshape(-1, BATCH)
    dstr = dstf.reshape(-1, BATCH)
    srcr64 = srcf.reshape(-1, 64)
    dstr64 = dstf.reshape(-1, 64)

    degparts = _sc_deg(dstr).reshape(2, n, 16)

    h1 = _tc_matmul1(x, W1)
    g1p, dinv16 = _tc_scale1(h1, degparts)

    agg1 = _sc_agg128(srcr64, dstr64, g1p).reshape(2, n, 128)

    w2p = jnp.pad(W2, ((0, 0), (0, 16 - dout)))
    g2p = _tc_layer2(agg1, dinv16, g1p, b1.reshape(1, din), w2p)

    agg2 = _sc_agg16(srcr, dstr, g2p).reshape(2, n, 16)

    b2r = jnp.pad(b2, (0, 16 - dout)).reshape(1, 16)
    out16 = _tc_final(agg2, dinv16, g2p, b2r)
    return out16[:, :dout]


# speedup vs baseline: 42.7622x; 1.0173x over previous
import functools

import jax
import jax.numpy as jnp
from jax import lax
from jax.experimental import pallas as pl
from jax.experimental.pallas import tpu as pltpu
from jax.experimental.pallas import tpu_sc as plsc

N_NODES = 10000
N_PAD = 10240
TILES = 32
BATCH = 128
NBT = 80
PER_TILE = NBT * BATCH
E_PAD = TILES * PER_TILE
NS = 16
ROWS_PER_SUB = N_PAD // NS
ROWS_OUT = N_NODES // NS

_SC_PARAMS = pltpu.CompilerParams(use_tc_tiling_on_sc=False)


def _make_sc_agg_ring(d, batch, nring, lead, chunks):
    nbt = PER_TILE // batch
    nc = nbt // chunks
    assert nc % nring == 0 and lead < nring
    assert ROWS_PER_SUB % batch == 0
    mesh = plsc.VectorSubcoreMesh(core_axis_name="c", subcore_axis_name="s")

    @functools.partial(
        pl.kernel,
        mesh=mesh,
        out_type=jax.ShapeDtypeStruct((2 * N_NODES, d), jnp.float32),
        scratch_types=[
            pltpu.VMEM((nc + lead, batch), jnp.int32),
            pltpu.VMEM((nc, batch), jnp.int32),
            pltpu.VMEM((nring, batch, d), jnp.float32),
            pltpu.VMEM_SHARED((N_PAD, d), jnp.float32),
        ] + [pltpu.SemaphoreType.DMA] * (2 * nring),
        compiler_params=_SC_PARAMS,
    )
    def agg(srcr_hbm, dstr_hbm, table_hbm, out_hbm,
            sidx, didx, gbuf, acc, *sems):
        gsems = sems[:nring]
        ssems = sems[nring:]
        c = lax.axis_index("c")
        s = lax.axis_index("s")
        wid = c * NS + s
        def zrow(r, carry):
            for k in range(d // 16):
                gbuf[0, r, pl.ds(k * 16, 16)] = jnp.zeros((16,), jnp.float32)
            return carry

        lax.fori_loop(0, batch, zrow, 0)
        for i in range(ROWS_PER_SUB // batch):
            pltpu.async_copy(
                gbuf.at[0],
                acc.at[pl.ds(s * ROWS_PER_SUB + i * batch, batch)], gsems[0])
        for i in range(ROWS_PER_SUB // batch):
            pltpu.make_async_copy(
                gbuf.at[0],
                acc.at[pl.ds(s * ROWS_PER_SUB, batch)], gsems[0]).wait()
        plsc.subcore_barrier()

        def visit(j, b):
            b2 = (b + lead) % nring
            pltpu.make_async_copy(table_hbm.at[sidx.at[b]], gbuf.at[b],
                                  gsems[b]).wait()
            pltpu.async_copy(gbuf.at[b], acc.at[didx.at[j]], ssems[b],
                             add=True)
            pltpu.async_copy(table_hbm.at[sidx.at[j + lead]], gbuf.at[b2],
                             gsems[b2])

        for ch in range(chunks):
            row0 = wid * nbt + ch * nc
            pltpu.sync_copy(srcr_hbm.at[pl.ds(row0, nc + lead)], sidx)
            pltpu.sync_copy(dstr_hbm.at[pl.ds(row0, nc)], didx)
            for jj in range(lead):
                pltpu.async_copy(table_hbm.at[sidx.at[jj]], gbuf.at[jj],
                                 gsems[jj])
            for jj in range(nring):
                b2 = (jj + lead) % nring
                if jj + lead >= nring:
                    pltpu.make_async_copy(gbuf.at[b2], acc.at[didx.at[0]],
                                          ssems[b2]).wait()
                visit(jj, jj)

            def body(g, carry):
                for bb in range(nring):
                    j = g * nring + bb
                    b2 = (bb + lead) % nring
                    pltpu.make_async_copy(gbuf.at[b2], acc.at[didx.at[0]],
                                          ssems[b2]).wait()
                    visit(j, bb)
                return carry

            lax.fori_loop(1, nc // nring, body, 0)
            for bb in range(lead, nring):
                pltpu.make_async_copy(gbuf.at[bb], acc.at[didx.at[0]],
                                      ssems[bb]).wait()
            for bb in range(lead):
                pltpu.make_async_copy(table_hbm.at[sidx.at[bb]], gbuf.at[bb],
                                      gsems[bb]).wait()
        plsc.subcore_barrier()
        pltpu.sync_copy(
            acc.at[pl.ds(s * ROWS_OUT, ROWS_OUT)],
            out_hbm.at[pl.ds(c * N_NODES + s * ROWS_OUT, ROWS_OUT)])

    return agg


def _make_sc_deg():
    mesh = plsc.VectorSubcoreMesh(core_axis_name="c", subcore_axis_name="s")

    @functools.partial(
        pl.kernel,
        mesh=mesh,
        out_type=jax.ShapeDtypeStruct((2 * N_NODES, 16), jnp.float32),
        scratch_types=[
            pltpu.VMEM((NBT, BATCH), jnp.int32),
            pltpu.VMEM((BATCH, 16), jnp.float32),
            pltpu.VMEM((BATCH, 16), jnp.float32),
            pltpu.VMEM_SHARED((N_PAD, 16), jnp.float32),
            pltpu.SemaphoreType.DMA,
        ],
        compiler_params=_SC_PARAMS,
    )
    def deg(dstr_hbm, out_hbm, didx, ones_v, zeros_v, acc, sem):
        c = lax.axis_index("c")
        s = lax.axis_index("s")
        wid = c * NS + s

        def fill(r, carry):
            ones_v[r, pl.ds(0, 16)] = jnp.ones((16,), jnp.float32)
            zeros_v[r, pl.ds(0, 16)] = jnp.zeros((16,), jnp.float32)
            return carry

        lax.fori_loop(0, BATCH, fill, 0)
        pltpu.sync_copy(dstr_hbm.at[pl.ds(wid * NBT, NBT)], didx)
        for i in range(ROWS_PER_SUB // BATCH):
            pltpu.async_copy(
                zeros_v,
                acc.at[pl.ds(s * ROWS_PER_SUB + i * BATCH, BATCH)], sem)
        for i in range(ROWS_PER_SUB // BATCH):
            pltpu.make_async_copy(
                zeros_v, acc.at[pl.ds(s * ROWS_PER_SUB, BATCH)], sem).wait()
        plsc.subcore_barrier()

        def fire(j, carry):
            pltpu.async_copy(ones_v, acc.at[didx.at[j]], sem, add=True)
            return carry

        lax.fori_loop(0, NBT, fire, 0)

        def drain(j, carry):
            pltpu.make_async_copy(ones_v, acc.at[didx.at[0]], sem).wait()
            return carry

        lax.fori_loop(0, NBT, drain, 0)
        plsc.subcore_barrier()
        pltpu.sync_copy(
            acc.at[pl.ds(s * ROWS_OUT, ROWS_OUT)],
            out_hbm.at[pl.ds(c * N_NODES + s * ROWS_OUT, ROWS_OUT)])

    return deg


_sc_deg = _make_sc_deg()
_sc_agg128 = _make_sc_agg_ring(128, batch=64, nring=5, lead=3, chunks=4)
_sc_agg16 = _make_sc_agg_ring(16, batch=128, nring=10, lead=5, chunks=1)

GRID = 5
RB = N_NODES // GRID


def _tc_layer1(x, w1, degparts):
    def body(x_ref, w_ref, dp_ref, g1_ref, dinv_ref):
        dinv = lax.rsqrt(dp_ref[0] + dp_ref[1] + 1.0)
        h = jnp.dot(x_ref[...], w_ref[...], preferred_element_type=jnp.float32)
        g1_ref[...] = dinv[:, 0:1] * h
        dinv_ref[...] = dinv

    return pl.pallas_call(
        body,
        grid=(GRID,),
        in_specs=[
            pl.BlockSpec((RB, 128), lambda i: (i, 0)),
            pl.BlockSpec((128, 128), lambda i: (0, 0)),
            pl.BlockSpec((2, RB, 16), lambda i: (0, i, 0)),
        ],
        out_specs=[
            pl.BlockSpec((RB, 128), lambda i: (i, 0)),
            pl.BlockSpec((RB, 16), lambda i: (i, 0)),
        ],
        out_shape=[
            jax.ShapeDtypeStruct((N_NODES, 128), jnp.float32),
            jax.ShapeDtypeStruct((N_NODES, 16), jnp.float32),
        ],
    )(x, w1, degparts)


def _tc_layer2(aggparts, dinv16, g1p, b1r, w2p):
    def body(ap_ref, dinv_ref, g1_ref, b_ref, w_ref, g2_ref):
        dinv = dinv_ref[...]
        tot = ap_ref[0] + ap_ref[1] + g1_ref[...]
        out1 = jnp.maximum(dinv[:, 0:1] * tot + b_ref[...], 0.0)
        g2_ref[...] = dinv * jnp.dot(out1, w_ref[...],
                                     preferred_element_type=jnp.float32)

    return pl.pallas_call(
        body,
        grid=(GRID,),
        in_specs=[
            pl.BlockSpec((2, RB, 128), lambda i: (0, i, 0)),
            pl.BlockSpec((RB, 16), lambda i: (i, 0)),
            pl.BlockSpec((RB, 128), lambda i: (i, 0)),
            pl.BlockSpec((1, 128), lambda i: (0, 0)),
            pl.BlockSpec((128, 16), lambda i: (0, 0)),
        ],
        out_specs=pl.BlockSpec((RB, 16), lambda i: (i, 0)),
        out_shape=jax.ShapeDtypeStruct((N_NODES, 16), jnp.float32),
    )(aggparts, dinv16, g1p, b1r, w2p)


def _tc_final(aggparts, dinv16, g2p, b2r):
    def body(ap_ref, dinv_ref, g2_ref, b_ref, o_ref):
        o_ref[...] = (dinv_ref[...] * (ap_ref[0] + ap_ref[1] + g2_ref[...])
                      + b_ref[...])

    return pl.pallas_call(
        body,
        grid=(GRID,),
        in_specs=[
            pl.BlockSpec((2, RB, 16), lambda i: (0, i, 0)),
            pl.BlockSpec((RB, 16), lambda i: (i, 0)),
            pl.BlockSpec((RB, 16), lambda i: (i, 0)),
            pl.BlockSpec((1, 16), lambda i: (0, 0)),
        ],
        out_specs=pl.BlockSpec((RB, 16), lambda i: (i, 0)),
        out_shape=jax.ShapeDtypeStruct((N_NODES, 16), jnp.float32),
    )(aggparts, dinv16, g2p, b2r)


def kernel(x, edge_index, W1, b1, W2, b2):
    n, din = x.shape
    dout = W2.shape[1]
    pad_e = E_PAD + 640 - edge_index.shape[1]
    padc = jnp.arange(pad_e, dtype=jnp.int32) % (N_PAD - n)
    srcf = jnp.concatenate([edge_index[0], padc])
    dstf = jnp.concatenate([edge_index[1], n + padc])
    srcr = srcf.reshape(-1, BATCH)
    dstr = dstf.reshape(-1, BATCH)
    srcr64 = srcf.reshape(-1, 64)
    dstr64 = dstf.reshape(-1, 64)

    degparts = _sc_deg(dstr).reshape(2, n, 16)

    g1p, dinv16 = _tc_layer1(x, W1, degparts)

    agg1 = _sc_agg128(srcr64, dstr64, g1p).reshape(2, n, 128)

    w2p = jnp.pad(W2, ((0, 0), (0, 16 - dout)))
    g2p = _tc_layer2(agg1, dinv16, g1p, b1.reshape(1, din), w2p)

    agg2 = _sc_agg16(srcr, dstr, g2p).reshape(2, n, 16)

    b2r = jnp.pad(b2, (0, 16 - dout)).reshape(1, 16)
    out16 = _tc_final(agg2, dinv16, g2p, b2r)
    return out16[:, :dout]
